# baseline (device time: 21702 ns/iter reference)
import jax
import jax.numpy as jnp
from jax import lax
from jax.experimental import pallas as pl
from jax.experimental.pallas import tpu as pltpu


def kernel(x):
    m, n = x.shape
    half = n // 2
    Q = m // 4
    R = Q // 2
    L = m // 4

    def body(x_hbm, out_ref, comm_ref, xin_ref, lbuf_ref, cin_sems, lsem,
             psig, sy, ry, sx, rx, sz, rz):
        my_x = lax.axis_index("x")
        my_y = lax.axis_index("y")
        my_z = lax.axis_index("z")
        peer_y = 1 - my_y

        xn = (1 - my_x, my_y, my_z)
        zn = (my_x, my_y, 1 - my_z)
        yp = (my_x, peer_y, my_z)

        q_me_early = 2 * my_z + my_x
        q_diag_early = 2 * (1 - my_z) + (1 - my_x)

        cin = []
        for k, quarter in enumerate((q_me_early, q_me_early, q_diag_early,
                                     q_diag_early)):
            row = quarter * Q + (k % 2) * R
            c = pltpu.make_async_copy(
                x_hbm.at[pl.ds(row, R), pl.ds(peer_y * half, half)],
                xin_ref.at[pl.ds(k * R, R), :],
                cin_sems.at[k],
            )
            c.start()
            cin.append(c)
        lcopy = pltpu.make_async_copy(
            x_hbm.at[:, pl.ds(my_y * half, half)], lbuf_ref, lsem
        )
        lcopy.start()

        barrier_sem = pltpu.get_barrier_semaphore()
        pl.semaphore_signal(
            barrier_sem, inc=1, device_id=yp,
            device_id_type=pl.DeviceIdType.MESH,
        )
        for nbr in (xn, zn):
            pl.semaphore_signal(
                psig, inc=1, device_id=nbr,
                device_id_type=pl.DeviceIdType.MESH,
            )
        pl.semaphore_wait(barrier_sem, 1)

        my_base = my_y * m
        peer_base = peer_y * m

        q_me = 2 * my_z + my_x
        q_diag = 2 * (1 - my_z) + (1 - my_x)

        y_rdmas = []
        for k, (quarter, cidx) in enumerate(
            [(q_me, 0), (q_me, 1), (q_diag, 0), (q_diag, 1)]
        ):
            row = quarter * Q + cidx * R
            cin[k].wait()
            comm_ref[pl.ds(k * R, R), :] = xin_ref[pl.ds(k * R, R), :].astype(
                jnp.bfloat16
            )
            r = pltpu.make_async_remote_copy(
                src_ref=comm_ref.at[pl.ds(k * R, R), :],
                dst_ref=out_ref.at[pl.ds(my_base + row, R), :],
                send_sem=sy.at[k],
                recv_sem=ry.at[k],
                device_id=yp,
                device_id_type=pl.DeviceIdType.MESH,
            )
            r.start()
            y_rdmas.append(r)

        pl.semaphore_wait(psig, 2)

        lcopy.wait()
        out_ref[pl.ds(my_base, L), :] = lbuf_ref[pl.ds(0, L), :].astype(
            jnp.bfloat16
        )

        fwd_rdmas = []
        for k in range(2):
            y_rdmas[k].wait_recv()
            rows = peer_base + q_me * Q + k * R
            for dst_dev, s_arr, r_arr in ((xn, sx, rx), (zn, sz, rz)):
                r = pltpu.make_async_remote_copy(
                    src_ref=out_ref.at[pl.ds(rows, R), :],
                    dst_ref=out_ref.at[pl.ds(rows, R), :],
                    send_sem=s_arr.at[k],
                    recv_sem=r_arr.at[k],
                    device_id=dst_dev,
                    device_id_type=pl.DeviceIdType.MESH,
                )
                r.start()
                fwd_rdmas.append(r)
            out_ref[pl.ds(my_base + (k + 1) * L, L), :] = lbuf_ref[
                pl.ds((k + 1) * L, L), :
            ].astype(jnp.bfloat16)

        out_ref[pl.ds(my_base + 3 * L, L), :] = lbuf_ref[
            pl.ds(3 * L, L), :
        ].astype(jnp.bfloat16)

        def wait_in(sem_arr, idx):
            r = pltpu.make_async_remote_copy(
                src_ref=comm_ref.at[pl.ds(0, R), :],
                dst_ref=comm_ref.at[pl.ds(0, R), :],
                send_sem=sy.at[0], recv_sem=sem_arr.at[idx],
                device_id=yp, device_id_type=pl.DeviceIdType.MESH,
            )
            r.wait_recv()

        y_rdmas[2].wait_recv()
        wait_in(rx, 0)
        wait_in(rz, 0)
        y_rdmas[3].wait_recv()
        wait_in(rx, 1)
        wait_in(rz, 1)

        for r in y_rdmas:
            r.wait_send()
        for r in fwd_rdmas:
            r.wait_send()

    return pl.pallas_call(
        body,
        out_shape=jax.ShapeDtypeStruct((2 * m, half), jnp.bfloat16),
        in_specs=[pl.BlockSpec(memory_space=pl.ANY)],
        out_specs=pl.BlockSpec(memory_space=pltpu.VMEM),
        scratch_shapes=[
            pltpu.VMEM((4 * R, half), jnp.bfloat16),
            pltpu.VMEM((4 * R, half), x.dtype),
            pltpu.VMEM((m, half), x.dtype),
            pltpu.SemaphoreType.DMA((4,)),
            pltpu.SemaphoreType.DMA,
            pltpu.SemaphoreType.REGULAR,
            pltpu.SemaphoreType.DMA((4,)),
            pltpu.SemaphoreType.DMA((4,)),
            pltpu.SemaphoreType.DMA((2,)),
            pltpu.SemaphoreType.DMA((2,)),
            pltpu.SemaphoreType.DMA((2,)),
            pltpu.SemaphoreType.DMA((2,)),
        ],
        compiler_params=pltpu.CompilerParams(collective_id=0),
    )(x)


# device time: 20658 ns/iter; 1.0505x vs baseline; 1.0505x over previous
import jax
import jax.numpy as jnp
from jax import lax
from jax.experimental import pallas as pl
from jax.experimental.pallas import tpu as pltpu


def kernel(x):
    m, n = x.shape
    half = n // 2
    Q = m // 4
    R = Q // 2
    L = m // 4

    def body(x_ref, out_ref, comm_ref, psig, sy, ry, sx, rx, sz, rz):
        my_x = lax.axis_index("x")
        my_y = lax.axis_index("y")
        my_z = lax.axis_index("z")
        peer_y = 1 - my_y

        xn = (1 - my_x, my_y, my_z)
        zn = (my_x, my_y, 1 - my_z)
        yp = (my_x, peer_y, my_z)

        barrier_sem = pltpu.get_barrier_semaphore()
        pl.semaphore_signal(
            barrier_sem, inc=1, device_id=yp,
            device_id_type=pl.DeviceIdType.MESH,
        )
        for nbr in (xn, zn):
            pl.semaphore_signal(
                psig, inc=1, device_id=nbr,
                device_id_type=pl.DeviceIdType.MESH,
            )
        pl.semaphore_wait(barrier_sem, 1)

        my_base = my_y * m
        peer_base = peer_y * m

        q_me = 2 * my_z + my_x
        q_diag = 2 * (1 - my_z) + (1 - my_x)

        y_rdmas = []
        for k, (quarter, cidx) in enumerate(
            [(q_me, 0), (q_me, 1), (q_diag, 0), (q_diag, 1)]
        ):
            row = quarter * Q + cidx * R
            comm_ref[pl.ds(k * R, R), :] = x_ref[
                pl.ds(row, R), pl.ds(peer_y * half, half)
            ].astype(jnp.bfloat16)
            r = pltpu.make_async_remote_copy(
                src_ref=comm_ref.at[pl.ds(k * R, R), :],
                dst_ref=out_ref.at[pl.ds(my_base + row, R), :],
                send_sem=sy.at[k],
                recv_sem=ry.at[k],
                device_id=yp,
                device_id_type=pl.DeviceIdType.MESH,
            )
            r.start()
            y_rdmas.append(r)

        pl.semaphore_wait(psig, 2)

        out_ref[pl.ds(my_base, L), :] = x_ref[
            pl.ds(0, L), pl.ds(my_y * half, half)
        ].astype(jnp.bfloat16)

        fwd_rdmas = []
        for k in range(2):
            y_rdmas[k].wait_recv()
            rows = peer_base + q_me * Q + k * R
            for dst_dev, s_arr, r_arr in ((xn, sx, rx), (zn, sz, rz)):
                r = pltpu.make_async_remote_copy(
                    src_ref=out_ref.at[pl.ds(rows, R), :],
                    dst_ref=out_ref.at[pl.ds(rows, R), :],
                    send_sem=s_arr.at[k],
                    recv_sem=r_arr.at[k],
                    device_id=dst_dev,
                    device_id_type=pl.DeviceIdType.MESH,
                )
                r.start()
                fwd_rdmas.append(r)
            out_ref[pl.ds(my_base + (k + 1) * L, L), :] = x_ref[
                pl.ds((k + 1) * L, L), pl.ds(my_y * half, half)
            ].astype(jnp.bfloat16)

        out_ref[pl.ds(my_base + 3 * L, L), :] = x_ref[
            pl.ds(3 * L, L), pl.ds(my_y * half, half)
        ].astype(jnp.bfloat16)

        def wait_in(sem_arr, idx):
            r = pltpu.make_async_remote_copy(
                src_ref=comm_ref.at[pl.ds(0, R), :],
                dst_ref=comm_ref.at[pl.ds(0, R), :],
                send_sem=sy.at[0], recv_sem=sem_arr.at[idx],
                device_id=yp, device_id_type=pl.DeviceIdType.MESH,
            )
            r.wait_recv()

        y_rdmas[2].wait_recv()
        wait_in(rx, 0)
        wait_in(rz, 0)
        y_rdmas[3].wait_recv()
        wait_in(rx, 1)
        wait_in(rz, 1)

        for r in y_rdmas:
            r.wait_send()
        for r in fwd_rdmas:
            r.wait_send()

    return pl.pallas_call(
        body,
        out_shape=jax.ShapeDtypeStruct((2 * m, half), jnp.bfloat16),
        in_specs=[pl.BlockSpec(memory_space=pltpu.VMEM)],
        out_specs=pl.BlockSpec(memory_space=pltpu.VMEM),
        scratch_shapes=[
            pltpu.VMEM((4 * R, half), jnp.bfloat16),
            pltpu.SemaphoreType.REGULAR,
            pltpu.SemaphoreType.DMA((4,)),
            pltpu.SemaphoreType.DMA((4,)),
            pltpu.SemaphoreType.DMA((2,)),
            pltpu.SemaphoreType.DMA((2,)),
            pltpu.SemaphoreType.DMA((2,)),
            pltpu.SemaphoreType.DMA((2,)),
        ],
        compiler_params=pltpu.CompilerParams(collective_id=0),
    )(x)
